# Initial kernel scaffold; baseline (speedup 1.0000x reference)
import jax
import jax.numpy as jnp
from jax.experimental import pallas as pl


def kernel(targets_1_pos, contexts_1_pos, contexts_0_pos_samples, W_hidden, W_output):
    raise NotImplementedError("write your pallas kernel here")


# speedup vs baseline: 33.2268x; 33.2268x over previous
import jax
import jax.numpy as jnp
from jax import lax
from jax.experimental import pallas as pl
from jax.experimental.pallas import tpu as pltpu
from jax.experimental.pallas import tpu_sc as plsc

_VOCAB = 100
_D = 128
_B = 16384
_K = 20
_VPAD = 128
_NC = 2
_NS = 16
_NW = _NC * _NS
_BPW = _B // _NW
_L = 16


def _matmul_body(wh_ref, wo_ref, s_ref):
    s_ref[...] = lax.dot_general(
        wh_ref[...], wo_ref[...],
        dimension_numbers=(((1,), (1,)), ((), ())),
        preferred_element_type=jnp.float32)


def _reduce_body(p_ref, q_ref, o_ref):
    p = p_ref[...]
    q = q_ref[...]
    lp = jnp.sum(jax.nn.log_sigmoid(p))
    ln = jnp.sum(jax.nn.log_sigmoid(-q))
    o_ref[...] = jnp.reshape(-(lp + ln) / _B, (1, 1))


def _sc_body(s_hbm, t_hbm, c_hbm, n_hbm, p_hbm, q_hbm,
             s_v, t_v, c_v, n_v, p_v, q_v):
    wid = lax.axis_index("s") * _NC + lax.axis_index("c")
    base = wid * _BPW
    pltpu.sync_copy(s_hbm, s_v)
    pltpu.sync_copy(t_hbm.at[pl.ds(base, _BPW)], t_v)
    pltpu.sync_copy(c_hbm.at[pl.ds(base, _BPW)], c_v)
    pltpu.sync_copy(n_hbm.at[pl.ds(base * _K, _BPW * _K)], n_v)
    iota = lax.iota(jnp.int32, _L)

    def step(j, carry):
        row0 = j * _L
        trow = t_v[pl.ds(row0, _L)] * _VPAD
        cv = c_v[pl.ds(row0, _L)]
        p_v[pl.ds(row0, _L)] = plsc.load_gather(s_v, [trow + cv])
        acc = jnp.zeros((_L,), jnp.float32)
        nbase = (row0 + iota) * _K
        for k in range(_K):
            nk = plsc.load_gather(n_v, [nbase + k])
            acc = acc + plsc.load_gather(s_v, [trow + nk])
        q_v[pl.ds(row0, _L)] = acc
        return carry

    lax.fori_loop(0, _BPW // _L, step, 0)
    pltpu.sync_copy(p_v, p_hbm.at[pl.ds(base, _BPW)])
    pltpu.sync_copy(q_v, q_hbm.at[pl.ds(base, _BPW)])


_sc_scores = pl.kernel(
    _sc_body,
    mesh=plsc.VectorSubcoreMesh(core_axis_name="c", subcore_axis_name="s"),
    compiler_params=pltpu.CompilerParams(needs_layout_passes=False),
    out_type=[jax.ShapeDtypeStruct((_B,), jnp.float32),
              jax.ShapeDtypeStruct((_B,), jnp.float32)],
    scratch_types=[
        pltpu.VMEM((_VPAD * _VPAD,), jnp.float32),
        pltpu.VMEM((_BPW,), jnp.int32),
        pltpu.VMEM((_BPW,), jnp.int32),
        pltpu.VMEM((_BPW * _K,), jnp.int32),
        pltpu.VMEM((_BPW,), jnp.float32),
        pltpu.VMEM((_BPW,), jnp.float32),
    ],
)


def kernel(targets_1_pos, contexts_1_pos, contexts_0_pos_samples, W_hidden, W_output):
    f32 = jnp.float32
    wh = jnp.zeros((_VPAD, _D), f32).at[:_VOCAB, :].set(W_hidden.astype(f32))
    wo = jnp.zeros((_VPAD, _D), f32).at[:_VOCAB, :].set(W_output.astype(f32))
    s_mat = pl.pallas_call(
        _matmul_body,
        out_shape=jax.ShapeDtypeStruct((_VPAD, _VPAD), f32),
    )(wh, wo)
    t = targets_1_pos.astype(jnp.int32)
    c = contexts_1_pos.astype(jnp.int32)
    n = contexts_0_pos_samples.astype(jnp.int32).reshape(-1)
    p, q = _sc_scores(s_mat.reshape(-1), t, c, n)
    out = pl.pallas_call(
        _reduce_body,
        out_shape=jax.ShapeDtypeStruct((1, 1), f32),
    )(p.reshape(_VPAD, _VPAD), q.reshape(_VPAD, _VPAD))
    return out.reshape(1)
